# Initial kernel scaffold; baseline (speedup 1.0000x reference)
import jax
import jax.numpy as jnp
from jax.experimental import pallas as pl


def kernel(features, edge_index, W1, b1, W2, b2):
    raise NotImplementedError("write your pallas kernel here")


# speedup vs baseline: 5.0560x; 5.0560x over previous
import functools

import jax
import jax.numpy as jnp
from jax import lax
from jax.experimental import pallas as pl
from jax.experimental.pallas import tpu as pltpu
from jax.experimental.pallas import tpu_sc as plsc

NC = 2
NS = 16
NW = NC * NS
L = 16


def _sc_edge_agg(n_nodes, d, n_edges, chunk, zrows, stage_vals=False):
    e_per_w = n_edges // NW
    n_chunks = e_per_w // chunk
    rows_per_tile = (n_nodes // NS) // 8 * 8
    tail = n_nodes - rows_per_tile * NS
    n_zcopy = rows_per_tile // zrows
    assert e_per_w * NW == n_edges and n_chunks * chunk == e_per_w
    assert n_zcopy * zrows == rows_per_tile and 0 <= tail <= zrows and tail % 8 == 0
    assert chunk % 8 == 0 and chunk <= 128 and d % L == 0

    mesh = plsc.VectorSubcoreMesh(core_axis_name="c", subcore_axis_name="s")

    scratch = [
        pltpu.VMEM((chunk,), jnp.int32),
        pltpu.VMEM((chunk,), jnp.int32),
        pltpu.VMEM((chunk, d), jnp.float32),
        pltpu.VMEM((zrows, d), jnp.float32),
        pltpu.VMEM_SHARED((n_nodes, d), jnp.float32),
        pltpu.SemaphoreType.DMA,
    ]
    if stage_vals:
        scratch.append(pltpu.VMEM_SHARED((n_nodes, d), jnp.float32))

    @functools.partial(
        pl.kernel,
        mesh=mesh,
        out_type=jax.ShapeDtypeStruct((NC, n_nodes, d), jnp.float32),
        scratch_types=scratch,
    )
    def agg(src_hbm, dst_hbm, vals_hbm, out_hbm,
            src_v, dst_v, rows_v, zero_v, acc_sh, sem, *maybe_vals_sh):
        cid = lax.axis_index("c")
        sid = lax.axis_index("s")
        wid = cid * NS + sid
        row0 = sid * rows_per_tile

        zvec = jnp.zeros((L,), jnp.float32)
        dl = d // L

        def zstore(i, carry):
            zero_v[i // dl, pl.ds((i % dl) * L, L)] = zvec
            return carry

        lax.fori_loop(0, zrows * dl, zstore, 0)

        def zcopy(j, carry):
            pltpu.sync_copy(zero_v, acc_sh.at[pl.ds(row0 + j * zrows, zrows)])
            return carry

        lax.fori_loop(0, n_zcopy, zcopy, 0)

        if tail:
            @pl.when(sid == NS - 1)
            def _():
                pltpu.sync_copy(zero_v.at[pl.ds(0, tail)],
                                acc_sh.at[pl.ds(n_nodes - tail, tail)])

        if stage_vals:
            vals_sh = maybe_vals_sh[0]
            pltpu.sync_copy(vals_hbm.at[pl.ds(row0, rows_per_tile)],
                            vals_sh.at[pl.ds(row0, rows_per_tile)])
            if tail:
                @pl.when(sid == NS - 1)
                def _():
                    pltpu.sync_copy(vals_hbm.at[pl.ds(n_nodes - tail, tail)],
                                    vals_sh.at[pl.ds(n_nodes - tail, tail)])
            gather_src = vals_sh
        else:
            gather_src = vals_hbm

        plsc.subcore_barrier()

        e_base = wid * e_per_w

        def body(ci, carry):
            e0 = e_base + ci * chunk
            pltpu.sync_copy(src_hbm.at[pl.ds(e0, chunk)], src_v)
            pltpu.sync_copy(dst_hbm.at[pl.ds(e0, chunk)], dst_v)
            pltpu.async_copy(gather_src.at[src_v], rows_v, sem).wait()
            pltpu.sync_copy(rows_v, acc_sh.at[dst_v], add=True)
            return carry

        lax.fori_loop(0, n_chunks, body, 0)
        plsc.subcore_barrier()

        pltpu.sync_copy(acc_sh.at[pl.ds(row0, rows_per_tile)],
                        out_hbm.at[cid, pl.ds(row0, rows_per_tile)])

        if tail:
            @pl.when(sid == NS - 1)
            def _():
                pltpu.sync_copy(acc_sh.at[pl.ds(n_nodes - tail, tail)],
                                out_hbm.at[cid, pl.ds(n_nodes - tail, tail)])

    return agg


def _mid_body(p0, p1, w1, b1, w2, out):
    agg = p0[...] + p1[...]
    h = jnp.maximum(
        jnp.dot(agg, w1[...], preferred_element_type=jnp.float32) + b1[...], 0.0)
    out[...] = jnp.dot(h, w2[...], preferred_element_type=jnp.float32)


def _fin_body(q0, q1, b2, out):
    out[...] = q0[...] + q1[...] + b2[...]


def kernel(features, edge_index, W1, b1, W2, b2):
    n, d = features.shape
    e = edge_index.shape[1]
    d2 = 128
    block = 1000

    src = edge_index[0]
    dst = edge_index[1]
    W2p = jnp.pad(W2, ((0, 0), (0, d2 - W2.shape[1])))
    b2p = jnp.pad(b2, (0, d2 - b2.shape[0])).reshape(1, d2)
    b1r = b1.reshape(1, d)

    p = _sc_edge_agg(n, d, e, 80, 48)(src, dst, features)

    hw2 = pl.pallas_call(
        _mid_body,
        grid=(n // block,),
        in_specs=[
            pl.BlockSpec((block, d), lambda i: (i, 0)),
            pl.BlockSpec((block, d), lambda i: (i, 0)),
            pl.BlockSpec((d, d), lambda i: (0, 0)),
            pl.BlockSpec((1, d), lambda i: (0, 0)),
            pl.BlockSpec((d, d2), lambda i: (0, 0)),
        ],
        out_specs=pl.BlockSpec((block, d2), lambda i: (i, 0)),
        out_shape=jax.ShapeDtypeStruct((n, d2), jnp.float32),
    )(p[0], p[1], W1, b1r, W2p)

    q = _sc_edge_agg(n, d2, e, 80, 48)(src, dst, hw2)

    logits16 = pl.pallas_call(
        _fin_body,
        grid=(n // block,),
        in_specs=[
            pl.BlockSpec((block, d2), lambda i: (i, 0)),
            pl.BlockSpec((block, d2), lambda i: (i, 0)),
            pl.BlockSpec((1, d2), lambda i: (0, 0)),
        ],
        out_specs=pl.BlockSpec((block, d2), lambda i: (i, 0)),
        out_shape=jax.ShapeDtypeStruct((n, d2), jnp.float32),
    )(q[0], q[1], b2p)

    return lax.slice(logits16, (0, 0), (n, 7))


# speedup vs baseline: 9.1681x; 1.8133x over previous
import functools

import jax
import jax.numpy as jnp
from jax import lax
from jax.experimental import pallas as pl
from jax.experimental.pallas import tpu as pltpu
from jax.experimental.pallas import tpu_sc as plsc

NC = 2
NS = 16
NW = NC * NS
L = 16


def _sc_edge_agg(n_nodes, d, n_edges, chunk, zrows, stage_vals=False):
    e_per_w = n_edges // NW
    n_chunks = e_per_w // chunk
    rows_per_tile = (n_nodes // NS) // 8 * 8
    tail = n_nodes - rows_per_tile * NS
    n_zcopy = rows_per_tile // zrows
    assert e_per_w * NW == n_edges and n_chunks * chunk == e_per_w
    assert n_zcopy * zrows == rows_per_tile and 0 <= tail <= zrows and tail % 8 == 0
    assert chunk % 8 == 0 and chunk <= 128 and d % L == 0

    mesh = plsc.VectorSubcoreMesh(core_axis_name="c", subcore_axis_name="s")

    scratch = [
        pltpu.VMEM((2, chunk), jnp.int32),
        pltpu.VMEM((2, chunk), jnp.int32),
        pltpu.VMEM((2, chunk, d), jnp.float32),
        pltpu.VMEM((zrows, d), jnp.float32),
        pltpu.VMEM_SHARED((n_nodes, d), jnp.float32),
        pltpu.SemaphoreType.DMA,
        pltpu.SemaphoreType.DMA,
        pltpu.SemaphoreType.DMA,
        pltpu.SemaphoreType.DMA,
    ]
    if stage_vals:
        scratch.append(pltpu.VMEM_SHARED((n_nodes, d), jnp.float32))

    @functools.partial(
        pl.kernel,
        mesh=mesh,
        out_type=jax.ShapeDtypeStruct((NC, n_nodes, d), jnp.float32),
        scratch_types=scratch,
    )
    def agg(src_hbm, dst_hbm, vals_hbm, out_hbm,
            sbuf, dbuf, rows_v, zero_v, acc_sh, sg0, sg1, si0, si1,
            *maybe_vals_sh):
        cid = lax.axis_index("c")
        sid = lax.axis_index("s")
        wid = cid * NS + sid
        row0 = sid * rows_per_tile

        zvec = jnp.zeros((L,), jnp.float32)
        dl = d // L

        def zstore(i, carry):
            zero_v[i // dl, pl.ds((i % dl) * L, L)] = zvec
            return carry

        lax.fori_loop(0, zrows * dl, zstore, 0)

        def zcopy(j, carry):
            pltpu.sync_copy(zero_v, acc_sh.at[pl.ds(row0 + j * zrows, zrows)])
            return carry

        lax.fori_loop(0, n_zcopy, zcopy, 0)

        if tail:
            @pl.when(sid == NS - 1)
            def _():
                pltpu.sync_copy(zero_v.at[pl.ds(0, tail)],
                                acc_sh.at[pl.ds(n_nodes - tail, tail)])

        if stage_vals:
            vals_sh = maybe_vals_sh[0]
            pltpu.sync_copy(vals_hbm.at[pl.ds(row0, rows_per_tile)],
                            vals_sh.at[pl.ds(row0, rows_per_tile)])
            if tail:
                @pl.when(sid == NS - 1)
                def _():
                    pltpu.sync_copy(vals_hbm.at[pl.ds(n_nodes - tail, tail)],
                                    vals_sh.at[pl.ds(n_nodes - tail, tail)])
            gather_src = vals_sh
        else:
            gather_src = vals_hbm

        plsc.subcore_barrier()

        e_base = wid * e_per_w
        sg = (sg0, sg1)
        si = (si0, si1)

        def idx_start(i, b):
            e0 = e_base + i * chunk
            pltpu.async_copy(src_hbm.at[pl.ds(e0, chunk)], sbuf.at[b], si[b])
            pltpu.async_copy(dst_hbm.at[pl.ds(e0, chunk)], dbuf.at[b], si[b])

        def idx_wait(b):
            pltpu.make_async_copy(src_hbm.at[pl.ds(0, chunk)], sbuf.at[b], si[b]).wait()
            pltpu.make_async_copy(dst_hbm.at[pl.ds(0, chunk)], dbuf.at[b], si[b]).wait()

        def g_start(b):
            pltpu.async_copy(gather_src.at[sbuf.at[b]], rows_v.at[b], sg[b])

        def g_wait(b):
            pltpu.make_async_copy(gather_src.at[sbuf.at[b]], rows_v.at[b], sg[b]).wait()

        idx_start(0, 0)
        idx_start(1, 1)
        idx_wait(0)
        g_start(0)

        def pair(k, carry):
            for b in (0, 1):
                i = 2 * k + b

                @pl.when(i < n_chunks)
                def _():
                    @pl.when(i + 1 < n_chunks)
                    def _():
                        idx_wait(1 - b)

                    g_wait(b)

                    @pl.when(i + 1 < n_chunks)
                    def _():
                        g_start(1 - b)

                    pltpu.sync_copy(rows_v.at[b], acc_sh.at[dbuf.at[b]], add=True)

                    @pl.when(i + 2 < n_chunks)
                    def _():
                        idx_start(i + 2, b)
            return carry

        lax.fori_loop(0, (n_chunks + 1) // 2, pair, 0)
        plsc.subcore_barrier()

        pltpu.sync_copy(acc_sh.at[pl.ds(row0, rows_per_tile)],
                        out_hbm.at[cid, pl.ds(row0, rows_per_tile)])

        if tail:
            @pl.when(sid == NS - 1)
            def _():
                pltpu.sync_copy(acc_sh.at[pl.ds(n_nodes - tail, tail)],
                                out_hbm.at[cid, pl.ds(n_nodes - tail, tail)])

    return agg


def _mid_body(p0, p1, w1, b1, w2, out):
    agg = p0[...] + p1[...]
    h = jnp.maximum(
        jnp.dot(agg, w1[...], preferred_element_type=jnp.float32) + b1[...], 0.0)
    out[...] = jnp.dot(h, w2[...], preferred_element_type=jnp.float32)


def _fin_body(q0, q1, b2, out):
    out[...] = q0[...] + q1[...] + b2[...]


def kernel(features, edge_index, W1, b1, W2, b2):
    n, d = features.shape
    e = edge_index.shape[1]
    d2 = 128
    block = 1000

    src = edge_index[0]
    dst = edge_index[1]
    W2p = jnp.pad(W2, ((0, 0), (0, d2 - W2.shape[1])))
    b2p = jnp.pad(b2, (0, d2 - b2.shape[0])).reshape(1, d2)
    b1r = b1.reshape(1, d)

    p = _sc_edge_agg(n, d, e, 80, 48)(src, dst, features)

    hw2 = pl.pallas_call(
        _mid_body,
        grid=(n // block,),
        in_specs=[
            pl.BlockSpec((block, d), lambda i: (i, 0)),
            pl.BlockSpec((block, d), lambda i: (i, 0)),
            pl.BlockSpec((d, d), lambda i: (0, 0)),
            pl.BlockSpec((1, d), lambda i: (0, 0)),
            pl.BlockSpec((d, d2), lambda i: (0, 0)),
        ],
        out_specs=pl.BlockSpec((block, d2), lambda i: (i, 0)),
        out_shape=jax.ShapeDtypeStruct((n, d2), jnp.float32),
    )(p[0], p[1], W1, b1r, W2p)

    q = _sc_edge_agg(n, d2, e, 80, 48)(src, dst, hw2)

    logits16 = pl.pallas_call(
        _fin_body,
        grid=(n // block,),
        in_specs=[
            pl.BlockSpec((block, d2), lambda i: (i, 0)),
            pl.BlockSpec((block, d2), lambda i: (i, 0)),
            pl.BlockSpec((1, d2), lambda i: (0, 0)),
        ],
        out_specs=pl.BlockSpec((block, d2), lambda i: (i, 0)),
        out_shape=jax.ShapeDtypeStruct((n, d2), jnp.float32),
    )(q[0], q[1], b2p)

    return lax.slice(logits16, (0, 0), (n, 7))


# speedup vs baseline: 9.5265x; 1.0391x over previous
import functools

import jax
import jax.numpy as jnp
from jax import lax
from jax.experimental import pallas as pl
from jax.experimental.pallas import tpu as pltpu
from jax.experimental.pallas import tpu_sc as plsc

NC = 2
NS = 16
NW = NC * NS
L = 16


def _sc_edge_agg(n_nodes, d, n_edges, chunk, zrows, dn=None):
    e_per_w = n_edges // NW
    n_chunks = e_per_w // chunk
    rows_per_tile = (n_nodes // NS) // 8 * 8
    tail = n_nodes - rows_per_tile * NS
    n_zcopy = rows_per_tile // zrows
    da = dn if dn is not None else d
    assert e_per_w * NW == n_edges and n_chunks * chunk == e_per_w
    assert n_zcopy * zrows == rows_per_tile and 0 <= tail <= zrows and tail % 8 == 0
    assert chunk % 8 == 0 and chunk <= 128 and d % L == 0 and da % L == 0

    mesh = plsc.VectorSubcoreMesh(core_axis_name="c", subcore_axis_name="s")

    scratch = [
        pltpu.VMEM((2, chunk), jnp.int32),
        pltpu.VMEM((2, chunk), jnp.int32),
        pltpu.VMEM((2, chunk, d), jnp.float32),
        pltpu.VMEM((zrows, da), jnp.float32),
        pltpu.VMEM_SHARED((n_nodes, da), jnp.float32),
        pltpu.SemaphoreType.DMA,
        pltpu.SemaphoreType.DMA,
        pltpu.SemaphoreType.DMA,
        pltpu.SemaphoreType.DMA,
    ]
    if dn is not None:
        scratch.append(pltpu.VMEM((2, chunk, dn), jnp.float32))

    @functools.partial(
        pl.kernel,
        mesh=mesh,
        out_type=jax.ShapeDtypeStruct((NC, n_nodes, da), jnp.float32),
        scratch_types=scratch,
    )
    def agg(src_hbm, dst_hbm, vals_hbm, out_hbm,
            sbuf, dbuf, rows_v, zero_v, acc_sh, sg0, sg1, si0, si1,
            *maybe_rows_n):
        cid = lax.axis_index("c")
        sid = lax.axis_index("s")
        wid = cid * NS + sid
        row0 = sid * rows_per_tile

        zvec = jnp.zeros((L,), jnp.float32)
        dl = da // L

        def zstore(i, carry):
            zero_v[i // dl, pl.ds((i % dl) * L, L)] = zvec
            return carry

        lax.fori_loop(0, zrows * dl, zstore, 0)

        def zcopy(j, carry):
            pltpu.sync_copy(zero_v, acc_sh.at[pl.ds(row0 + j * zrows, zrows)])
            return carry

        lax.fori_loop(0, n_zcopy, zcopy, 0)

        if tail:
            @pl.when(sid == NS - 1)
            def _():
                pltpu.sync_copy(zero_v.at[pl.ds(0, tail)],
                                acc_sh.at[pl.ds(n_nodes - tail, tail)])

        gather_src = vals_hbm
        plsc.subcore_barrier()

        e_base = wid * e_per_w
        sg = (sg0, sg1)
        si = (si0, si1)

        def idx_start(i, b):
            e0 = e_base + i * chunk
            pltpu.async_copy(src_hbm.at[pl.ds(e0, chunk)], sbuf.at[b], si[b])
            pltpu.async_copy(dst_hbm.at[pl.ds(e0, chunk)], dbuf.at[b], si[b])

        def idx_wait(b):
            pltpu.make_async_copy(src_hbm.at[pl.ds(0, chunk)], sbuf.at[b], si[b]).wait()
            pltpu.make_async_copy(dst_hbm.at[pl.ds(0, chunk)], dbuf.at[b], si[b]).wait()

        def g_start(b):
            pltpu.async_copy(gather_src.at[sbuf.at[b]], rows_v.at[b], sg[b])

        def g_wait(b):
            pltpu.make_async_copy(gather_src.at[sbuf.at[b]], rows_v.at[b], sg[b]).wait()

        idx_start(0, 0)
        idx_start(1, 1)
        idx_wait(0)
        g_start(0)

        def pair(k, carry):
            for b in (0, 1):
                i = 2 * k + b

                @pl.when(i < n_chunks)
                def _():
                    @pl.when(i + 1 < n_chunks)
                    def _():
                        idx_wait(1 - b)

                    g_wait(b)

                    @pl.when(i + 1 < n_chunks)
                    def _():
                        g_start(1 - b)

                    if dn is None:
                        pltpu.sync_copy(rows_v.at[b], acc_sh.at[dbuf.at[b]],
                                        add=True)
                    else:
                        rows_n = maybe_rows_n[0]

                        def ebody(j, carry):
                            for c in range(dn // L):
                                rows_n[b, j, pl.ds(c * L, L)] = (
                                    rows_v[b, j, pl.ds(c * L, L)])
                            return carry

                        lax.fori_loop(0, chunk, ebody, 0)
                        pltpu.sync_copy(rows_n.at[b], acc_sh.at[dbuf.at[b]],
                                        add=True)

                    @pl.when(i + 2 < n_chunks)
                    def _():
                        idx_start(i + 2, b)
            return carry

        lax.fori_loop(0, (n_chunks + 1) // 2, pair, 0)
        plsc.subcore_barrier()

        pltpu.sync_copy(acc_sh.at[pl.ds(row0, rows_per_tile)],
                        out_hbm.at[cid, pl.ds(row0, rows_per_tile)])

        if tail:
            @pl.when(sid == NS - 1)
            def _():
                pltpu.sync_copy(acc_sh.at[pl.ds(n_nodes - tail, tail)],
                                out_hbm.at[cid, pl.ds(n_nodes - tail, tail)])

    return agg


def _mid_body(p0, p1, w1, b1, w2, out):
    agg = p0[0] + p1[0]
    h = jnp.maximum(
        jnp.dot(agg, w1[...], preferred_element_type=jnp.float32) + b1[...], 0.0)
    out[...] = jnp.dot(h, w2[...], preferred_element_type=jnp.float32)


def _fin_body(q0, q1, b2, out):
    out[...] = q0[0] + q1[0] + b2[...]


def kernel(features, edge_index, W1, b1, W2, b2):
    n, d = features.shape
    e = edge_index.shape[1]
    d2 = 128
    dn = 16
    block = 1000

    src = edge_index[0]
    dst = edge_index[1]
    W2p = jnp.pad(W2, ((0, 0), (0, d2 - W2.shape[1])))
    b2p = jnp.pad(b2, (0, d2 - b2.shape[0])).reshape(1, d2)
    b1r = b1.reshape(1, d)

    p = _sc_edge_agg(n, d, e, 80, 48)(src, dst, features)

    hw2 = pl.pallas_call(
        _mid_body,
        grid=(n // block,),
        in_specs=[
            pl.BlockSpec((1, block, d), lambda i: (0, i, 0)),
            pl.BlockSpec((1, block, d), lambda i: (1, i, 0)),
            pl.BlockSpec((d, d), lambda i: (0, 0)),
            pl.BlockSpec((1, d), lambda i: (0, 0)),
            pl.BlockSpec((d, d2), lambda i: (0, 0)),
        ],
        out_specs=pl.BlockSpec((block, d2), lambda i: (i, 0)),
        out_shape=jax.ShapeDtypeStruct((n, d2), jnp.float32),
    )(p, p, W1, b1r, W2p)

    q = _sc_edge_agg(n, d2, e, 80, 48)(src, dst, hw2)

    logits16 = pl.pallas_call(
        _fin_body,
        grid=(n // block,),
        in_specs=[
            pl.BlockSpec((1, block, d2), lambda i: (0, i, 0)),
            pl.BlockSpec((1, block, d2), lambda i: (1, i, 0)),
            pl.BlockSpec((1, d2), lambda i: (0, 0)),
        ],
        out_specs=pl.BlockSpec((block, d2), lambda i: (i, 0)),
        out_shape=jax.ShapeDtypeStruct((n, d2), jnp.float32),
    )(q, q, b2p)

    return lax.slice(logits16, (0, 0), (n, 7))
